# baseline (speedup 1.0000x reference)
import functools

import jax
import jax.numpy as jnp
import numpy as np
from jax import lax
from jax.experimental import pallas as pl
from jax.experimental.pallas import tpu as pltpu
from jax.experimental.pallas import tpu_sc as plsc

B = 16384
D = 128
H = 128
NC = 2
NS = 16
NW = NC * NS
SLICES = (B,)
C = 128
NSLOT = 2
AHEAD = 1


def _gather_mul_body(bpw, offset, uid_hbm, iid_hbm, utab_hbm, itab_hbm, out_hbm,
                     uidx, iidx, *rest):
    nchunk = bpw // C
    ubufs = rest[0:NSLOT]
    ibufs = rest[NSLOT:2 * NSLOT]
    rest = rest[2 * NSLOT:]
    sem_idx = rest[0]
    gsems = rest[1:1 + NSLOT]
    wsems = rest[1 + NSLOT:]
    wid = lax.axis_index("s") * NC + lax.axis_index("c")
    base = offset + wid * bpw

    cu_idx = pltpu.make_async_copy(uid_hbm.at[pl.ds(base, bpw)], uidx, sem_idx)
    ci_idx = pltpu.make_async_copy(iid_hbm.at[pl.ds(base, bpw)], iidx, sem_idx)
    cu_idx.start()
    ci_idx.start()
    cu_idx.wait()
    ci_idx.wait()

    def fire_gather(c):
        s = c % NSLOT
        g_u = pltpu.make_async_copy(
            utab_hbm.at[uidx.at[pl.ds(c * C, C)]], ubufs[s], gsems[s])
        g_i = pltpu.make_async_copy(
            itab_hbm.at[iidx.at[pl.ds(c * C, C)]], ibufs[s], gsems[s])
        g_u.start()
        g_i.start()
        return g_u, g_i

    writes = [None] * NSLOT
    pend = [fire_gather(c) for c in range(min(AHEAD, nchunk))]
    for c in range(nchunk):
        s = c % NSLOT
        pend[c][0].wait()
        pend[c][1].wait()

        urows, irows = ubufs[s], ibufs[s]

        def mul_row(i, carry):
            for j in range(D // 16):
                sl = pl.ds(j * 16, 16)
                urows[i, sl] = urows[i, sl] * irows[i, sl]
            return carry

        lax.fori_loop(0, C, mul_row, 0)
        w = pltpu.make_async_copy(
            urows, out_hbm.at[pl.ds(base - offset + c * C, C)], wsems[s])
        w.start()
        writes[s] = w
        nx = c + AHEAD
        if nx < nchunk:
            if writes[nx % NSLOT] is not None:
                writes[nx % NSLOT].wait()
                writes[nx % NSLOT] = None
            pend.append(fire_gather(nx))
    for w in writes:
        if w is not None:
            w.wait()


@functools.cache
def _gather_mul_fn(n_rows, offset):
    mesh = plsc.VectorSubcoreMesh(core_axis_name="c", subcore_axis_name="s")
    return pl.kernel(
        functools.partial(_gather_mul_body, n_rows // NW, offset),
        mesh=mesh,
        out_type=jax.ShapeDtypeStruct((n_rows, D), jnp.float32),
        scratch_types=(
            [pltpu.VMEM((n_rows // NW,), jnp.int32)] * 2
            + [pltpu.VMEM((C, D), jnp.float32)] * (2 * NSLOT)
            + [pltpu.SemaphoreType.DMA] * (1 + 2 * NSLOT)
        ),
    )


def _mlp_body(x_ref, w1t_ref, b1_ref, w2_ref, b2_ref, o_ref):
    xt = x_ref[...].T
    ht = jnp.dot(w1t_ref[...], xt, preferred_element_type=jnp.float32)
    ht = jnp.maximum(ht + b1_ref[...], 0.0)
    z = jnp.dot(w2_ref[...], ht, preferred_element_type=jnp.float32) + b2_ref[0]
    o_ref[...] = 1.0 / (1.0 + jnp.exp(-z))


def _mlp(x, W1t, b1c, W2r, b2):
    n_rows = x.shape[0]
    BT = 4096
    return pl.pallas_call(
        _mlp_body,
        grid=(n_rows // BT,),
        in_specs=[
            pl.BlockSpec((BT, D), lambda i: (i, 0)),
            pl.BlockSpec((D, H), lambda i: (0, 0)),
            pl.BlockSpec((H, 1), lambda i: (0, 0)),
            pl.BlockSpec((1, H), lambda i: (0, 0)),
            pl.BlockSpec(memory_space=pltpu.SMEM),
        ],
        out_specs=pl.BlockSpec((1, BT), lambda i: (0, i)),
        out_shape=jax.ShapeDtypeStruct((1, n_rows), jnp.float32),
    )(x, W1t, b1c, W2r, b2)


def kernel(user_id, item_id, user_table, item_table, W1, b1, W2, b2):
    uid = user_id.reshape(B).astype(jnp.int32)
    iid = item_id.reshape(B).astype(jnp.int32)
    W1t = W1.T
    b1c = b1.reshape(H, 1)
    W2r = W2.reshape(1, H)
    outs = []
    off = 0
    for n_rows in SLICES:
        x_s = _gather_mul_fn(n_rows, off)(uid, iid, user_table, item_table)
        outs.append(_mlp(x_s, W1t, b1c, W2r, b2))
        off += n_rows
    out = outs[0] if len(SLICES) == 1 else jnp.concatenate(outs, axis=1)
    return out.reshape(B, 1, 1)


# speedup vs baseline: 1.0496x; 1.0496x over previous
import functools

import jax
import jax.numpy as jnp
import numpy as np
from jax import lax
from jax.experimental import pallas as pl
from jax.experimental.pallas import tpu as pltpu
from jax.experimental.pallas import tpu_sc as plsc

B = 16384
D = 128
H = 128
NC = 2
NS = 16
NW = NC * NS
SLICES = (B,)
C = 128
NSLOT = 2
AHEAD = 1


def _gather_mul_body(bpw, offset, uid_hbm, iid_hbm, utab_hbm, itab_hbm, out_hbm,
                     uidx, iidx, *rest):
    nchunk = bpw // C
    ubufs = rest[0:NSLOT]
    ibufs = rest[NSLOT:2 * NSLOT]
    rest = rest[2 * NSLOT:]
    sem_idx = rest[0]
    gsems = rest[1:1 + NSLOT]
    wsems = rest[1 + NSLOT:]
    wid = lax.axis_index("s") * NC + lax.axis_index("c")
    base = offset + wid * bpw

    cu_idx = pltpu.make_async_copy(uid_hbm.at[pl.ds(base, bpw)], uidx, sem_idx)
    ci_idx = pltpu.make_async_copy(iid_hbm.at[pl.ds(base, bpw)], iidx, sem_idx)
    cu_idx.start()
    ci_idx.start()
    cu_idx.wait()
    ci_idx.wait()

    def fire_gather(c):
        s = c % NSLOT
        g_u = pltpu.make_async_copy(
            utab_hbm.at[uidx.at[pl.ds(c * C, C)]], ubufs[s], gsems[s])
        g_i = pltpu.make_async_copy(
            itab_hbm.at[iidx.at[pl.ds(c * C, C)]], ibufs[s], gsems[s])
        g_u.start()
        g_i.start()
        return g_u, g_i

    writes = [None] * NSLOT
    pend = [fire_gather(c) for c in range(min(AHEAD, nchunk))]
    for c in range(nchunk):
        s = c % NSLOT
        nx = c + AHEAD
        if nx < nchunk:
            if writes[nx % NSLOT] is not None:
                writes[nx % NSLOT].wait()
                writes[nx % NSLOT] = None
            pend.append(fire_gather(nx))
        pend[c][0].wait()
        pend[c][1].wait()

        urows, irows = ubufs[s], ibufs[s]

        def mul_row(i, carry):
            for j in range(D // 16):
                sl = pl.ds(j * 16, 16)
                urows[i, sl] = urows[i, sl] * irows[i, sl]
            return carry

        lax.fori_loop(0, C, mul_row, 0)
        w = pltpu.make_async_copy(
            urows, out_hbm.at[pl.ds(base - offset + c * C, C)], wsems[s])
        w.start()
        writes[s] = w
    for w in writes:
        if w is not None:
            w.wait()


@functools.cache
def _gather_mul_fn(n_rows, offset):
    mesh = plsc.VectorSubcoreMesh(core_axis_name="c", subcore_axis_name="s")
    return pl.kernel(
        functools.partial(_gather_mul_body, n_rows // NW, offset),
        mesh=mesh,
        out_type=jax.ShapeDtypeStruct((n_rows, D), jnp.float32),
        scratch_types=(
            [pltpu.VMEM((n_rows // NW,), jnp.int32)] * 2
            + [pltpu.VMEM((C, D), jnp.float32)] * (2 * NSLOT)
            + [pltpu.SemaphoreType.DMA] * (1 + 2 * NSLOT)
        ),
    )


def _mlp_body(x_ref, w1t_ref, b1_ref, w2_ref, b2_ref, o_ref):
    xt = x_ref[...].T
    ht = jnp.dot(w1t_ref[...], xt, preferred_element_type=jnp.float32)
    ht = jnp.maximum(ht + b1_ref[...], 0.0)
    z = jnp.dot(w2_ref[...], ht, preferred_element_type=jnp.float32) + b2_ref[0]
    o_ref[...] = 1.0 / (1.0 + jnp.exp(-z))


def _mlp(x, W1t, b1c, W2r, b2):
    n_rows = x.shape[0]
    BT = 2048
    return pl.pallas_call(
        _mlp_body,
        grid=(n_rows // BT,),
        in_specs=[
            pl.BlockSpec((BT, D), lambda i: (i, 0)),
            pl.BlockSpec((D, H), lambda i: (0, 0)),
            pl.BlockSpec((H, 1), lambda i: (0, 0)),
            pl.BlockSpec((1, H), lambda i: (0, 0)),
            pl.BlockSpec(memory_space=pltpu.SMEM),
        ],
        out_specs=pl.BlockSpec((1, BT), lambda i: (0, i)),
        out_shape=jax.ShapeDtypeStruct((1, n_rows), jnp.float32),
    )(x, W1t, b1c, W2r, b2)


def kernel(user_id, item_id, user_table, item_table, W1, b1, W2, b2):
    uid = user_id.reshape(B).astype(jnp.int32)
    iid = item_id.reshape(B).astype(jnp.int32)
    W1t = W1.T
    b1c = b1.reshape(H, 1)
    W2r = W2.reshape(1, H)
    outs = []
    off = 0
    for n_rows in SLICES:
        x_s = _gather_mul_fn(n_rows, off)(uid, iid, user_table, item_table)
        outs.append(_mlp(x_s, W1t, b1c, W2r, b2))
        off += n_rows
    out = outs[0] if len(SLICES) == 1 else jnp.concatenate(outs, axis=1)
    return out.reshape(B, 1, 1)
